# Initial kernel scaffold; baseline (speedup 1.0000x reference)
import jax
import jax.numpy as jnp
from jax.experimental import pallas as pl


def kernel(x):
    raise NotImplementedError("write your pallas kernel here")


# speedup vs baseline: 1.3857x; 1.3857x over previous
import jax
import jax.numpy as jnp
from jax import lax
from jax.experimental import pallas as pl
from jax.experimental.pallas import tpu as pltpu
from jax.experimental.pallas import tpu_sc as plsc

_MASK_RATIO = 0.75
_LANES = 16


def _sc_gather_and_mask(x_flat, gids, restore_flat, *, rows, d, tokens,
                        len_keep):
    info = plsc.get_sparse_core_info()
    nw = info.num_cores * info.num_subcores
    assert rows % nw == 0 and tokens % nw == 0
    rpw = rows // nw
    mpw = tokens // nw
    nch = 4
    assert rpw % nch == 0
    ch = rpw // nch
    assert ch <= 128 and mpw % _LANES == 0
    mesh = plsc.VectorSubcoreMesh(core_axis_name="c", subcore_axis_name="s")

    def body(x_hbm, gid_hbm, restore_hbm, vis_hbm, mask_hbm,
             idx_v, buf0, buf1, restore_v, mask_v,
             sem_g0, sem_g1, sem_o0, sem_o1):
        cid = lax.axis_index("c")
        sid = lax.axis_index("s")
        wid = sid * info.num_cores + cid
        base = wid * rpw
        pltpu.sync_copy(gid_hbm.at[pl.ds(base, rpw)], idx_v)

        bufs = (buf0, buf1)
        gsems = (sem_g0, sem_g1)
        osems = (sem_o0, sem_o1)
        out_pending = [None, None]

        def start_gather(ci):
            b = ci % 2
            if out_pending[b] is not None:
                out_pending[b].wait()
                out_pending[b] = None
            return pltpu.async_copy(
                x_hbm.at[idx_v.at[pl.ds(ci * ch, ch)]], bufs[b], gsems[b])

        prev = start_gather(0)

        mbase = wid * mpw
        pltpu.sync_copy(restore_hbm.at[pl.ds(mbase, mpw)], restore_v)
        lk = jnp.full((_LANES,), len_keep, jnp.int32)
        ones = jnp.full((_LANES,), 1.0, jnp.float32)
        zeros = jnp.zeros((_LANES,), jnp.float32)

        def mstep(i, carry):
            off = pl.multiple_of(i * _LANES, _LANES)
            r = restore_v[pl.ds(off, _LANES)]
            mask_v[pl.ds(off, _LANES)] = jnp.where(r >= lk, ones, zeros)
            return carry

        lax.fori_loop(0, mpw // _LANES, mstep, 0)
        pltpu.sync_copy(mask_v, mask_hbm.at[pl.ds(mbase, mpw)])

        for ci in range(nch):
            nxt = start_gather(ci + 1) if ci + 1 < nch else None
            prev.wait()
            b = ci % 2
            out_pending[b] = pltpu.async_copy(
                bufs[b], vis_hbm.at[pl.ds(base + ci * ch, ch)], osems[b])
            prev = nxt
        for h in out_pending:
            if h is not None:
                h.wait()

    kern = pl.kernel(
        body,
        out_type=(
            jax.ShapeDtypeStruct((rows, d), jnp.float32),
            jax.ShapeDtypeStruct((tokens,), jnp.float32),
        ),
        mesh=mesh,
        scratch_types=(
            pltpu.VMEM((rpw,), jnp.int32),
            pltpu.VMEM((ch, d), jnp.float32),
            pltpu.VMEM((ch, d), jnp.float32),
            pltpu.VMEM((mpw,), jnp.int32),
            pltpu.VMEM((mpw,), jnp.float32),
            pltpu.SemaphoreType.DMA,
            pltpu.SemaphoreType.DMA,
            pltpu.SemaphoreType.DMA,
            pltpu.SemaphoreType.DMA,
        ),
    )
    return kern(x_flat, gids, restore_flat)


def kernel(x):
    b, n, d = x.shape
    len_keep = int(n * (1 - _MASK_RATIO))
    noise = jax.random.uniform(jax.random.key(42), (b, n), dtype=jnp.float32)
    ids_shuffle = jnp.argsort(noise, axis=1)
    ids_restore = jnp.argsort(ids_shuffle, axis=1)
    ids_keep = ids_shuffle[:, :len_keep]
    gids = (ids_keep.astype(jnp.int32)
            + (jnp.arange(b, dtype=jnp.int32) * n)[:, None]).reshape(-1)
    vis_flat, mask_flat = _sc_gather_and_mask(
        x.reshape(b * n, d), gids,
        ids_restore.reshape(-1).astype(jnp.int32),
        rows=b * len_keep, d=d, tokens=b * n, len_keep=len_keep)
    return (vis_flat.reshape(b, len_keep, d), mask_flat.reshape(b, n),
            ids_restore, ids_keep)
